# baseline (speedup 1.0000x reference)
import functools

import jax
import jax.numpy as jnp
from jax.experimental import pallas as pl
from jax.experimental.pallas import tpu as pltpu

_TM = 8192
_TC = 128
_LANE = 128
_N_OUT = 3


def _linear_t_kernel(x_ref, w_ref, o_ref):
    w = w_ref[...].astype(jnp.bfloat16)
    for c in range(_TM // _TC):
        xc = x_ref[c * _TC:(c + 1) * _TC, :].astype(jnp.bfloat16)
        yc = jnp.dot(xc, w, preferred_element_type=jnp.float32)
        st = yc[:, :8].T
        o_ref[:, c * _TC:(c + 1) * _TC] = st[:_N_OUT, :]


@jax.jit
def kernel(representation, w_p, b_p):
    x = representation.astype(jnp.float32)
    B, D = x.shape
    grid = (pl.cdiv(B, _TM),)
    yt = pl.pallas_call(
        _linear_t_kernel,
        out_shape=jax.ShapeDtypeStruct((_N_OUT, B), jnp.float32),
        grid=grid,
        in_specs=[
            pl.BlockSpec((_TM, D), lambda i: (i, 0)),
            pl.BlockSpec((D, _LANE), lambda i: (0, 0)),
        ],
        out_specs=pl.BlockSpec((_N_OUT, _TM), lambda i: (0, i)),
        compiler_params=pltpu.CompilerParams(
            dimension_semantics=("parallel",)),
        cost_estimate=pl.CostEstimate(
            flops=2 * B * D * _LANE,
            transcendentals=0,
            bytes_accessed=(B * D + D * _LANE + B * _N_OUT) * 4,
        ),
    )(x, w_p)
    return yt.T + b_p[0:1, :_N_OUT]


# speedup vs baseline: 1.0417x; 1.0417x over previous
import jax
import jax.numpy as jnp
from jax.experimental import pallas as pl
from jax.experimental.pallas import tpu as pltpu

_TC = 128
_N_OUT = 3
_OPAD = 8
_BUF_ROWS = 8192
_TILE_ROWS = (1024, 1024, 2048, 4096, 8192, 8192, 8192)


def _ramp_kernel(x_hbm, w_ref, b_ref, o_ref, x_buf, sems):
    n = len(_TILE_ROWS)
    starts = []
    r = 0
    for rs in _TILE_ROWS:
        starts.append(r)
        r += rs

    def copy(t):
        r0, rs = starts[t], _TILE_ROWS[t]
        return pltpu.make_async_copy(
            x_hbm.at[pl.ds(r0, rs), :],
            x_buf.at[t % 2, pl.ds(0, rs), :],
            sems.at[t])

    w = w_ref[...].astype(jnp.bfloat16)
    b8 = b_ref[0:1, :_OPAD]
    copy(0).start()
    copy(1).start()
    for t in range(n):
        copy(t).wait()
        r0, rs = starts[t], _TILE_ROWS[t]
        for c in range(rs // _TC):
            xc = x_buf[t % 2, c * _TC:(c + 1) * _TC, :].astype(jnp.bfloat16)
            yc = jnp.dot(xc, w, preferred_element_type=jnp.float32)
            s = yc[:, :_OPAD] + b8
            st = s.T
            o_ref[:, r0 + c * _TC: r0 + (c + 1) * _TC] = st[:_N_OUT, :]
        if t + 2 < n:
            copy(t + 2).start()


@jax.jit
def kernel(representation, w_p, b_p):
    x = representation.astype(jnp.float32)
    B, D = x.shape
    yt = pl.pallas_call(
        _ramp_kernel,
        out_shape=jax.ShapeDtypeStruct((_N_OUT, B), jnp.float32),
        in_specs=[
            pl.BlockSpec(memory_space=pltpu.MemorySpace.HBM),
            pl.BlockSpec(memory_space=pltpu.MemorySpace.VMEM),
            pl.BlockSpec(memory_space=pltpu.MemorySpace.VMEM),
        ],
        out_specs=pl.BlockSpec(memory_space=pltpu.MemorySpace.VMEM),
        scratch_shapes=[
            pltpu.VMEM((2, _BUF_ROWS, 256), jnp.float32),
            pltpu.SemaphoreType.DMA((len(_TILE_ROWS),)),
        ],
        cost_estimate=pl.CostEstimate(
            flops=2 * B * D * 128,
            transcendentals=0,
            bytes_accessed=(B * D + D * 128 + 128 + B * _N_OUT) * 4,
        ),
    )(x, w_p, b_p)
    return yt.T


# speedup vs baseline: 1.2020x; 1.1539x over previous
import functools

import jax
import jax.numpy as jnp
from jax.experimental import pallas as pl
from jax.experimental.pallas import tpu as pltpu

_TM = 8192
_TC = 128
_LANE = 128
_N_OUT = 3
_OPAD = 8


def _linear_t_kernel(xlo_ref, xhi_ref, w_ref, b_ref, o_ref):
    w = w_ref[...].astype(jnp.bfloat16)
    b8 = b_ref[0:1, :_OPAD]
    half = _TM // 2
    for h, x_ref in ((0, xlo_ref), (1, xhi_ref)):
        for c in range(half // _TC):
            xc = x_ref[c * _TC:(c + 1) * _TC, :].astype(jnp.bfloat16)
            yc = jnp.dot(xc, w, preferred_element_type=jnp.float32)
            s = yc[:, :_OPAD] + b8
            st = s.T
            col = h * half + c * _TC
            o_ref[:, col:col + _TC] = st[:_N_OUT, :]


@jax.jit
def kernel(representation, w_p, b_p):
    x = representation.astype(jnp.float32)
    B, D = x.shape
    grid = (pl.cdiv(B, _TM),)
    yt = pl.pallas_call(
        _linear_t_kernel,
        out_shape=jax.ShapeDtypeStruct((_N_OUT, B), jnp.float32),
        grid=grid,
        in_specs=[
            pl.BlockSpec((_TM // 2, D), lambda i: (2 * i, 0)),
            pl.BlockSpec((_TM // 2, D), lambda i: (2 * i + 1, 0)),
            pl.BlockSpec((D, _LANE), lambda i: (0, 0)),
            pl.BlockSpec((1, _LANE), lambda i: (0, 0)),
        ],
        out_specs=pl.BlockSpec((_N_OUT, _TM), lambda i: (0, i)),
        compiler_params=pltpu.CompilerParams(
            dimension_semantics=("parallel",)),
        cost_estimate=pl.CostEstimate(
            flops=2 * B * D * _LANE,
            transcendentals=0,
            bytes_accessed=(B * D + D * _LANE + _LANE + B * _N_OUT) * 4,
        ),
    )(x, x, w_p, b_p)
    return yt.T
